# Initial kernel scaffold; baseline (speedup 1.0000x reference)
import jax
import jax.numpy as jnp
from jax.experimental import pallas as pl


def kernel(x, state, conv_state, attention_norm_weight, ffn_norm_weight, in_proj_qkv, in_proj_z, in_proj_a, in_proj_b, conv1d_weight, dt_bias, A_log, norm_weight, out_proj, router_weight, expert_gate_up, expert_down, shared_gate, shared_up, shared_down, shared_expert_gate, linear_layer_idx):
    raise NotImplementedError("write your pallas kernel here")


# speedup vs baseline: 14.5339x; 14.5339x over previous
import functools

import jax
import jax.numpy as jnp
from jax.experimental import pallas as pl
from jax.experimental.pallas import tpu as pltpu

B, S, DIM = 1, 2048, 1024
NK, DK, NV, DV = 4, 64, 8, 64
KEY_DIM, VAL_DIM = NK * DK, NV * DV
CONV_DIM, KCONV = 2 * KEY_DIM + VAL_DIM, 4
E, TOPK, MOE_I, SHARED_I = 8, 2, 512, 512
EPS = 1e-6
RB = 256
CHUNK = 64
NCHUNK = S // CHUNK


def _silu(x):
    return x * jax.nn.sigmoid(x)


def _rms(x, w1p):
    return x * jax.lax.rsqrt(jnp.mean(x * x, axis=-1, keepdims=True) + EPS) * w1p


def _prologue_kernel(x_ref, wall_ref, anw_ref, dtb_ref, alog_ref,
                     qkv_ref, z_ref, g_ref, beta_ref):
    h = _rms(x_ref[...], 1.0 + anw_ref[...])
    y = jnp.dot(h, wall_ref[...], preferred_element_type=jnp.float32)
    qkv_ref[...] = y[:, :CONV_DIM]
    z_ref[...] = y[:, CONV_DIM:CONV_DIM + VAL_DIM]
    a = y[:, CONV_DIM + VAL_DIM:CONV_DIM + VAL_DIM + NV]
    b = y[:, CONV_DIM + VAL_DIM + NV:]
    g_ref[...] = -jnp.exp(alog_ref[...]) * jax.nn.softplus(a + dtb_ref[...])
    beta_ref[...] = jax.nn.sigmoid(b)


def _conv_kernel(xpad_ref, wt_ref, qn_ref, kn_ref, v_ref):
    acc = xpad_ref[5:5 + S, :] * wt_ref[0:1, :]
    for j in range(1, KCONV):
        acc = acc + xpad_ref[5 + j:5 + j + S, :] * wt_ref[j:j + 1, :]
    y = _silu(acc)
    for hh in range(NK):
        qh = y[:, hh * DK:(hh + 1) * DK]
        kh = y[:, KEY_DIM + hh * DK:KEY_DIM + (hh + 1) * DK]
        qn_ref[:, hh * DK:(hh + 1) * DK] = qh * jax.lax.rsqrt(
            jnp.sum(qh * qh, axis=-1, keepdims=True) + 1e-6)
        kn_ref[:, hh * DK:(hh + 1) * DK] = kh * jax.lax.rsqrt(
            jnp.sum(kh * kh, axis=-1, keepdims=True) + 1e-6)
    v_ref[...] = y[:, 2 * KEY_DIM:]


def _deltanet_kernel(qn_ref, kn_ref, v_ref, g_ref, beta_ref, s0_ref,
                     o_ref, sout_ref, s_scr):
    i = pl.program_id(0)

    @pl.when(i == 0)
    def _():
        s_scr[...] = s0_ref[...]

    row = jax.lax.broadcasted_iota(jnp.int32, (CHUNK, CHUNK), 0)
    col = jax.lax.broadcasted_iota(jnp.int32, (CHUNK, CHUNK), 1)
    incl = row >= col
    strict = row > col
    ltri = jnp.where(incl, 1.0, 0.0).astype(jnp.float32)
    eye = jnp.where(row == col, 1.0, 0.0).astype(jnp.float32)

    g_all = jnp.dot(ltri, g_ref[...], preferred_element_type=jnp.float32)

    for h in range(NV):
        hk = h // (NV // NK)
        q = qn_ref[:, hk * DK:(hk + 1) * DK]
        k = kn_ref[:, hk * DK:(hk + 1) * DK]
        v = v_ref[:, h * DV:(h + 1) * DV]
        gc = g_all[:, h:h + 1]
        beta = beta_ref[:, h:h + 1]
        s0 = s_scr[h * DK:(h + 1) * DK, :]

        dmat = jnp.exp(jnp.where(incl, gc - gc.T, -1e30))
        kkt = jnp.dot(k, k.T, preferred_element_type=jnp.float32)
        n = jnp.where(strict, -(beta * dmat * kkt), 0.0)
        p = eye + n
        q2 = jnp.dot(n, n, preferred_element_type=jnp.float32)
        for _ in range(5):
            p = p + jnp.dot(q2, p, preferred_element_type=jnp.float32)
            q2 = jnp.dot(q2, q2, preferred_element_type=jnp.float32)
        lam = jnp.exp(gc)
        w = (beta * lam) * k
        u = beta * v - jnp.dot(w, s0, preferred_element_type=jnp.float32)
        delta = jnp.dot(p, u, preferred_element_type=jnp.float32)

        qkt = jnp.dot(q, k.T, preferred_element_type=jnp.float32)
        o = jnp.dot(lam * q, s0, preferred_element_type=jnp.float32) + \
            jnp.dot(dmat * qkt, delta, preferred_element_type=jnp.float32)
        o_ref[:, h * DV:(h + 1) * DV] = o

        glast = gc[CHUNK - 1:CHUNK, :]
        kdec = jnp.exp(glast - gc) * k
        s1 = jnp.exp(glast) * s0 + jax.lax.dot_general(
            kdec, delta, (((0,), (0,)), ((), ())),
            preferred_element_type=jnp.float32)
        s_scr[h * DK:(h + 1) * DK, :] = s1

    @pl.when(i == NCHUNK - 1)
    def _():
        sout_ref[...] = s_scr[...]


def _epilogue_kernel(o_ref, z_ref, x_ref, nw_ref, fnw_ref, wout_ref,
                     rwt_ref, sg_ref, su_ref, sd_ref, seg_ref,
                     x2b_ref, h2_ref, cw_ref):
    o = _rms(o_ref[...], 1.0 + nw_ref[...]) * _silu(z_ref[...])
    attn = jnp.dot(o, wout_ref[...], preferred_element_type=jnp.float32)
    x2 = x_ref[...] + attn
    h2 = _rms(x2, 1.0 + fnw_ref[...])
    h2_ref[...] = h2

    logits = jnp.dot(h2, rwt_ref[...], preferred_element_type=jnp.float32)
    m = jnp.max(logits, axis=-1, keepdims=True)
    ex = jnp.exp(logits - m)
    probs = ex / jnp.sum(ex, axis=-1, keepdims=True)

    lane = jax.lax.broadcasted_iota(jnp.int32, probs.shape, 1)
    m0 = jnp.max(probs, axis=-1, keepdims=True)
    i0 = jnp.min(jnp.where(probs == m0, lane, E), axis=-1, keepdims=True)
    sel0 = lane == i0
    p2 = jnp.where(sel0, -jnp.inf, probs)
    m1 = jnp.max(p2, axis=-1, keepdims=True)
    i1 = jnp.min(jnp.where(p2 == m1, lane, E), axis=-1, keepdims=True)
    sel1 = lane == i1
    denom = m0 + m1
    cw_ref[...] = (jnp.where(sel0, m0, 0.0) + jnp.where(sel1, m1, 0.0)) / denom

    sact = _silu(jnp.dot(h2, sg_ref[...], preferred_element_type=jnp.float32)) * \
        jnp.dot(h2, su_ref[...], preferred_element_type=jnp.float32)
    shared = jnp.dot(sact, sd_ref[...], preferred_element_type=jnp.float32)
    gate = jax.nn.sigmoid(
        jnp.dot(h2, seg_ref[...], preferred_element_type=jnp.float32))
    x2b_ref[...] = x2 + gate * shared


def _expert_kernel(h2_ref, egu_ref, ed_ref, cw_ref, x2b_ref, out_ref):
    e = pl.program_id(1)
    gu = jax.lax.dot_general(h2_ref[...], egu_ref[0],
                             (((1,), (1,)), ((), ())),
                             preferred_element_type=jnp.float32)
    act = _silu(gu[:, :MOE_I]) * gu[:, MOE_I:]
    down = jax.lax.dot_general(act, ed_ref[0],
                               (((1,), (1,)), ((), ())),
                               preferred_element_type=jnp.float32)
    lane = jax.lax.broadcasted_iota(jnp.int32, (RB, E), 1)
    we = jnp.sum(jnp.where(lane == e, cw_ref[...], 0.0),
                 axis=-1, keepdims=True)
    contrib = we * down

    @pl.when(e == 0)
    def _():
        out_ref[...] = x2b_ref[...] + contrib

    @pl.when(e > 0)
    def _():
        out_ref[...] = out_ref[...] + contrib


def kernel(x, state, conv_state, attention_norm_weight, ffn_norm_weight,
           in_proj_qkv, in_proj_z, in_proj_a, in_proj_b, conv1d_weight,
           dt_bias, A_log, norm_weight, out_proj, router_weight,
           expert_gate_up, expert_down, shared_gate, shared_up, shared_down,
           shared_expert_gate, linear_layer_idx):
    f32 = jnp.float32
    x2d = x[0]
    w_all = jnp.concatenate([in_proj_qkv, in_proj_z, in_proj_a, in_proj_b],
                            axis=1)
    anw = attention_norm_weight[None, :]
    dtb = dt_bias[None, :]
    alog = A_log[None, :]

    nrow = S // RB
    qkv, z, g, beta = pl.pallas_call(
        _prologue_kernel,
        grid=(nrow,),
        in_specs=[
            pl.BlockSpec((RB, DIM), lambda i: (i, 0)),
            pl.BlockSpec((DIM, w_all.shape[1]), lambda i: (0, 0)),
            pl.BlockSpec((1, DIM), lambda i: (0, 0)),
            pl.BlockSpec((1, NV), lambda i: (0, 0)),
            pl.BlockSpec((1, NV), lambda i: (0, 0)),
        ],
        out_specs=[
            pl.BlockSpec((RB, CONV_DIM), lambda i: (i, 0)),
            pl.BlockSpec((RB, VAL_DIM), lambda i: (i, 0)),
            pl.BlockSpec((RB, NV), lambda i: (i, 0)),
            pl.BlockSpec((RB, NV), lambda i: (i, 0)),
        ],
        out_shape=[
            jax.ShapeDtypeStruct((S, CONV_DIM), f32),
            jax.ShapeDtypeStruct((S, VAL_DIM), f32),
            jax.ShapeDtypeStruct((S, NV), f32),
            jax.ShapeDtypeStruct((S, NV), f32),
        ],
    )(x2d, w_all, anw, dtb, alog)

    pre = jnp.zeros((8, CONV_DIM), f32).at[5:8, :].set(conv_state[0].T)
    xpad = jnp.concatenate([pre, qkv], axis=0)
    wt = conv1d_weight.T

    qn, kn, v = pl.pallas_call(
        _conv_kernel,
        in_specs=[
            pl.BlockSpec((S + 8, CONV_DIM), lambda: (0, 0)),
            pl.BlockSpec((KCONV, CONV_DIM), lambda: (0, 0)),
        ],
        out_specs=[
            pl.BlockSpec((S, KEY_DIM), lambda: (0, 0)),
            pl.BlockSpec((S, KEY_DIM), lambda: (0, 0)),
            pl.BlockSpec((S, VAL_DIM), lambda: (0, 0)),
        ],
        out_shape=[
            jax.ShapeDtypeStruct((S, KEY_DIM), f32),
            jax.ShapeDtypeStruct((S, KEY_DIM), f32),
            jax.ShapeDtypeStruct((S, VAL_DIM), f32),
        ],
    )(xpad, wt)

    s0 = state[0].reshape(NV * DK, DV)
    o, snew = pl.pallas_call(
        _deltanet_kernel,
        grid=(NCHUNK,),
        in_specs=[
            pl.BlockSpec((CHUNK, KEY_DIM), lambda i: (i, 0)),
            pl.BlockSpec((CHUNK, KEY_DIM), lambda i: (i, 0)),
            pl.BlockSpec((CHUNK, VAL_DIM), lambda i: (i, 0)),
            pl.BlockSpec((CHUNK, NV), lambda i: (i, 0)),
            pl.BlockSpec((CHUNK, NV), lambda i: (i, 0)),
            pl.BlockSpec((NV * DK, DV), lambda i: (0, 0)),
        ],
        out_specs=[
            pl.BlockSpec((CHUNK, VAL_DIM), lambda i: (i, 0)),
            pl.BlockSpec((NV * DK, DV), lambda i: (0, 0)),
        ],
        out_shape=[
            jax.ShapeDtypeStruct((S, VAL_DIM), f32),
            jax.ShapeDtypeStruct((NV * DK, DV), f32),
        ],
        scratch_shapes=[pltpu.VMEM((NV * DK, DV), f32)],
        compiler_params=pltpu.CompilerParams(
            dimension_semantics=("arbitrary",)),
    )(qn, kn, v, g, beta, s0)

    x2b, h2, cw = pl.pallas_call(
        _epilogue_kernel,
        grid=(nrow,),
        in_specs=[
            pl.BlockSpec((RB, VAL_DIM), lambda i: (i, 0)),
            pl.BlockSpec((RB, VAL_DIM), lambda i: (i, 0)),
            pl.BlockSpec((RB, DIM), lambda i: (i, 0)),
            pl.BlockSpec((1, VAL_DIM), lambda i: (0, 0)),
            pl.BlockSpec((1, DIM), lambda i: (0, 0)),
            pl.BlockSpec((VAL_DIM, DIM), lambda i: (0, 0)),
            pl.BlockSpec((DIM, E), lambda i: (0, 0)),
            pl.BlockSpec((DIM, SHARED_I), lambda i: (0, 0)),
            pl.BlockSpec((DIM, SHARED_I), lambda i: (0, 0)),
            pl.BlockSpec((SHARED_I, DIM), lambda i: (0, 0)),
            pl.BlockSpec((DIM, 1), lambda i: (0, 0)),
        ],
        out_specs=[
            pl.BlockSpec((RB, DIM), lambda i: (i, 0)),
            pl.BlockSpec((RB, DIM), lambda i: (i, 0)),
            pl.BlockSpec((RB, E), lambda i: (i, 0)),
        ],
        out_shape=[
            jax.ShapeDtypeStruct((S, DIM), f32),
            jax.ShapeDtypeStruct((S, DIM), f32),
            jax.ShapeDtypeStruct((S, E), f32),
        ],
    )(o, z, x2d, norm_weight[None, :], ffn_norm_weight[None, :], out_proj,
      router_weight.T, shared_gate, shared_up, shared_down,
      shared_expert_gate)

    x3 = pl.pallas_call(
        _expert_kernel,
        grid=(nrow, E),
        in_specs=[
            pl.BlockSpec((RB, DIM), lambda i, e: (i, 0)),
            pl.BlockSpec((1, 2 * MOE_I, DIM), lambda i, e: (e, 0, 0)),
            pl.BlockSpec((1, DIM, MOE_I), lambda i, e: (e, 0, 0)),
            pl.BlockSpec((RB, E), lambda i, e: (i, 0)),
            pl.BlockSpec((RB, DIM), lambda i, e: (i, 0)),
        ],
        out_specs=pl.BlockSpec((RB, DIM), lambda i, e: (i, 0)),
        out_shape=jax.ShapeDtypeStruct((S, DIM), f32),
        compiler_params=pltpu.CompilerParams(
            dimension_semantics=("parallel", "arbitrary")),
    )(h2, expert_gate_up, expert_down, cw, x2b)

    new_conv_state = qkv[S - (KCONV - 1):, :].T[None, :, :]
    return (x3[None, :, :], snew.reshape(1, NV, DK, DV), new_conv_state)


# speedup vs baseline: 14.6841x; 1.0103x over previous
import functools

import jax
import jax.numpy as jnp
from jax.experimental import pallas as pl
from jax.experimental.pallas import tpu as pltpu

B, S, DIM = 1, 2048, 1024
NK, DK, NV, DV = 4, 64, 8, 64
KEY_DIM, VAL_DIM = NK * DK, NV * DV
CONV_DIM, KCONV = 2 * KEY_DIM + VAL_DIM, 4
E, TOPK, MOE_I, SHARED_I = 8, 2, 512, 512
EPS = 1e-6
RB = 256
CHUNK = 64
NCHUNK = S // CHUNK


def _silu(x):
    return x * jax.nn.sigmoid(x)


def _rms(x, w1p):
    return x * jax.lax.rsqrt(jnp.mean(x * x, axis=-1, keepdims=True) + EPS) * w1p


def _prologue_kernel(x_ref, wall_ref, anw_ref, dtb_ref, alog_ref,
                     qkv_ref, z_ref, g_ref, beta_ref):
    h = _rms(x_ref[...], 1.0 + anw_ref[...])
    y = jnp.dot(h.astype(jnp.bfloat16), wall_ref[...],
                preferred_element_type=jnp.float32)
    qkv_ref[...] = y[:, :CONV_DIM]
    z_ref[...] = y[:, CONV_DIM:CONV_DIM + VAL_DIM]
    a = y[:, CONV_DIM + VAL_DIM:CONV_DIM + VAL_DIM + NV]
    b = y[:, CONV_DIM + VAL_DIM + NV:]
    g_ref[...] = -jnp.exp(alog_ref[...]) * jax.nn.softplus(a + dtb_ref[...])
    beta_ref[...] = jax.nn.sigmoid(b)


def _conv_kernel(xpad_ref, wt_ref, qn_ref, kn_ref, v_ref):
    acc = xpad_ref[5:5 + S, :] * wt_ref[0:1, :]
    for j in range(1, KCONV):
        acc = acc + xpad_ref[5 + j:5 + j + S, :] * wt_ref[j:j + 1, :]
    y = _silu(acc)
    for hh in range(NK):
        qh = y[:, hh * DK:(hh + 1) * DK]
        kh = y[:, KEY_DIM + hh * DK:KEY_DIM + (hh + 1) * DK]
        qn_ref[:, hh * DK:(hh + 1) * DK] = qh * jax.lax.rsqrt(
            jnp.sum(qh * qh, axis=-1, keepdims=True) + 1e-6)
        kn_ref[:, hh * DK:(hh + 1) * DK] = kh * jax.lax.rsqrt(
            jnp.sum(kh * kh, axis=-1, keepdims=True) + 1e-6)
    v_ref[...] = y[:, 2 * KEY_DIM:]


def _deltanet_kernel(qn_ref, kn_ref, v_ref, g_ref, beta_ref, s0_ref,
                     o_ref, sout_ref, s_scr):
    i = pl.program_id(0)

    @pl.when(i == 0)
    def _():
        s_scr[...] = s0_ref[...]

    row = jax.lax.broadcasted_iota(jnp.int32, (CHUNK, CHUNK), 0)
    col = jax.lax.broadcasted_iota(jnp.int32, (CHUNK, CHUNK), 1)
    incl = row >= col
    strict = row > col
    ltri = jnp.where(incl, 1.0, 0.0).astype(jnp.float32)
    eye = jnp.where(row == col, 1.0, 0.0).astype(jnp.float32)

    g_all = jnp.dot(ltri, g_ref[...], preferred_element_type=jnp.float32)

    for h in range(NV):
        hk = h // (NV // NK)
        q = qn_ref[:, hk * DK:(hk + 1) * DK]
        k = kn_ref[:, hk * DK:(hk + 1) * DK]
        v = v_ref[:, h * DV:(h + 1) * DV]
        gc = g_all[:, h:h + 1]
        beta = beta_ref[:, h:h + 1]
        s0 = s_scr[h * DK:(h + 1) * DK, :]

        dmat = jnp.exp(jnp.where(incl, gc - gc.T, -1e30))
        kkt = jnp.dot(k, k.T, preferred_element_type=jnp.float32)
        n = jnp.where(strict, -(beta * dmat * kkt), 0.0)
        p = eye + n
        q2 = jnp.dot(n, n, preferred_element_type=jnp.float32)
        for _ in range(5):
            p = p + jnp.dot(q2, p, preferred_element_type=jnp.float32)
            q2 = jnp.dot(q2, q2, preferred_element_type=jnp.float32)
        lam = jnp.exp(gc)
        w = (beta * lam) * k
        u = beta * v - jnp.dot(w, s0, preferred_element_type=jnp.float32)
        delta = jnp.dot(p, u, preferred_element_type=jnp.float32)

        qkt = jnp.dot(q, k.T, preferred_element_type=jnp.float32)
        o = jnp.dot(lam * q, s0, preferred_element_type=jnp.float32) + \
            jnp.dot(dmat * qkt, delta, preferred_element_type=jnp.float32)
        o_ref[:, h * DV:(h + 1) * DV] = o

        glast = gc[CHUNK - 1:CHUNK, :]
        kdec = jnp.exp(glast - gc) * k
        s1 = jnp.exp(glast) * s0 + jax.lax.dot_general(
            kdec, delta, (((0,), (0,)), ((), ())),
            preferred_element_type=jnp.float32)
        s_scr[h * DK:(h + 1) * DK, :] = s1

    @pl.when(i == NCHUNK - 1)
    def _():
        sout_ref[...] = s_scr[...]


def _epilogue_kernel(o_ref, z_ref, x_ref, nw_ref, fnw_ref, wout_ref,
                     rwt_ref, sg_ref, su_ref, sd_ref, seg_ref,
                     x2b_ref, h2_ref, cw_ref):
    o = _rms(o_ref[...], 1.0 + nw_ref[...]) * _silu(z_ref[...])
    attn = jnp.dot(o.astype(jnp.bfloat16), wout_ref[...],
                   preferred_element_type=jnp.float32)
    x2 = x_ref[...] + attn
    h2 = _rms(x2, 1.0 + fnw_ref[...])
    h2_ref[...] = h2
    h2b = h2.astype(jnp.bfloat16)

    logits = jnp.dot(h2, rwt_ref[...], preferred_element_type=jnp.float32)
    m = jnp.max(logits, axis=-1, keepdims=True)
    ex = jnp.exp(logits - m)
    probs = ex / jnp.sum(ex, axis=-1, keepdims=True)

    lane = jax.lax.broadcasted_iota(jnp.int32, probs.shape, 1)
    m0 = jnp.max(probs, axis=-1, keepdims=True)
    i0 = jnp.min(jnp.where(probs == m0, lane, E), axis=-1, keepdims=True)
    sel0 = lane == i0
    p2 = jnp.where(sel0, -jnp.inf, probs)
    m1 = jnp.max(p2, axis=-1, keepdims=True)
    i1 = jnp.min(jnp.where(p2 == m1, lane, E), axis=-1, keepdims=True)
    sel1 = lane == i1
    denom = m0 + m1
    cw_ref[...] = (jnp.where(sel0, m0, 0.0) + jnp.where(sel1, m1, 0.0)) / denom

    sact = _silu(jnp.dot(h2b, sg_ref[...], preferred_element_type=jnp.float32)) * \
        jnp.dot(h2b, su_ref[...], preferred_element_type=jnp.float32)
    shared = jnp.dot(sact.astype(jnp.bfloat16), sd_ref[...],
                     preferred_element_type=jnp.float32)
    gate = jax.nn.sigmoid(
        jnp.dot(h2, seg_ref[...], preferred_element_type=jnp.float32))
    x2b_ref[...] = x2 + gate * shared


def _expert_kernel(h2_ref, egu_ref, ed_ref, cw_ref, x2b_ref, out_ref):
    e = pl.program_id(1)
    gu = jax.lax.dot_general(h2_ref[...].astype(jnp.bfloat16), egu_ref[0],
                             (((1,), (1,)), ((), ())),
                             preferred_element_type=jnp.float32)
    act = _silu(gu[:, :MOE_I]) * gu[:, MOE_I:]
    down = jax.lax.dot_general(act.astype(jnp.bfloat16), ed_ref[0],
                               (((1,), (1,)), ((), ())),
                               preferred_element_type=jnp.float32)
    lane = jax.lax.broadcasted_iota(jnp.int32, (RB, E), 1)
    we = jnp.sum(jnp.where(lane == e, cw_ref[...], 0.0),
                 axis=-1, keepdims=True)
    contrib = we * down

    @pl.when(e == 0)
    def _():
        out_ref[...] = x2b_ref[...] + contrib

    @pl.when(e > 0)
    def _():
        out_ref[...] = out_ref[...] + contrib


def kernel(x, state, conv_state, attention_norm_weight, ffn_norm_weight,
           in_proj_qkv, in_proj_z, in_proj_a, in_proj_b, conv1d_weight,
           dt_bias, A_log, norm_weight, out_proj, router_weight,
           expert_gate_up, expert_down, shared_gate, shared_up, shared_down,
           shared_expert_gate, linear_layer_idx):
    f32 = jnp.float32
    x2d = x[0]
    w_all = jnp.concatenate([in_proj_qkv, in_proj_z, in_proj_a, in_proj_b],
                            axis=1).astype(jnp.bfloat16)
    anw = attention_norm_weight[None, :]
    dtb = dt_bias[None, :]
    alog = A_log[None, :]

    nrow = S // RB
    qkv, z, g, beta = pl.pallas_call(
        _prologue_kernel,
        grid=(nrow,),
        in_specs=[
            pl.BlockSpec((RB, DIM), lambda i: (i, 0)),
            pl.BlockSpec((DIM, w_all.shape[1]), lambda i: (0, 0)),
            pl.BlockSpec((1, DIM), lambda i: (0, 0)),
            pl.BlockSpec((1, NV), lambda i: (0, 0)),
            pl.BlockSpec((1, NV), lambda i: (0, 0)),
        ],
        out_specs=[
            pl.BlockSpec((RB, CONV_DIM), lambda i: (i, 0)),
            pl.BlockSpec((RB, VAL_DIM), lambda i: (i, 0)),
            pl.BlockSpec((RB, NV), lambda i: (i, 0)),
            pl.BlockSpec((RB, NV), lambda i: (i, 0)),
        ],
        out_shape=[
            jax.ShapeDtypeStruct((S, CONV_DIM), f32),
            jax.ShapeDtypeStruct((S, VAL_DIM), f32),
            jax.ShapeDtypeStruct((S, NV), f32),
            jax.ShapeDtypeStruct((S, NV), f32),
        ],
    )(x2d, w_all, anw, dtb, alog)

    pre = jnp.zeros((8, CONV_DIM), f32).at[5:8, :].set(conv_state[0].T)
    xpad = jnp.concatenate([pre, qkv], axis=0)
    wt = conv1d_weight.T

    qn, kn, v = pl.pallas_call(
        _conv_kernel,
        in_specs=[
            pl.BlockSpec((S + 8, CONV_DIM), lambda: (0, 0)),
            pl.BlockSpec((KCONV, CONV_DIM), lambda: (0, 0)),
        ],
        out_specs=[
            pl.BlockSpec((S, KEY_DIM), lambda: (0, 0)),
            pl.BlockSpec((S, KEY_DIM), lambda: (0, 0)),
            pl.BlockSpec((S, VAL_DIM), lambda: (0, 0)),
        ],
        out_shape=[
            jax.ShapeDtypeStruct((S, KEY_DIM), f32),
            jax.ShapeDtypeStruct((S, KEY_DIM), f32),
            jax.ShapeDtypeStruct((S, VAL_DIM), f32),
        ],
    )(xpad, wt)

    s0 = state[0].reshape(NV * DK, DV)
    o, snew = pl.pallas_call(
        _deltanet_kernel,
        grid=(NCHUNK,),
        in_specs=[
            pl.BlockSpec((CHUNK, KEY_DIM), lambda i: (i, 0)),
            pl.BlockSpec((CHUNK, KEY_DIM), lambda i: (i, 0)),
            pl.BlockSpec((CHUNK, VAL_DIM), lambda i: (i, 0)),
            pl.BlockSpec((CHUNK, NV), lambda i: (i, 0)),
            pl.BlockSpec((CHUNK, NV), lambda i: (i, 0)),
            pl.BlockSpec((NV * DK, DV), lambda i: (0, 0)),
        ],
        out_specs=[
            pl.BlockSpec((CHUNK, VAL_DIM), lambda i: (i, 0)),
            pl.BlockSpec((NV * DK, DV), lambda i: (0, 0)),
        ],
        out_shape=[
            jax.ShapeDtypeStruct((S, VAL_DIM), f32),
            jax.ShapeDtypeStruct((NV * DK, DV), f32),
        ],
        scratch_shapes=[pltpu.VMEM((NV * DK, DV), f32)],
        compiler_params=pltpu.CompilerParams(
            dimension_semantics=("arbitrary",)),
    )(qn, kn, v, g, beta, s0)

    x2b, h2, cw = pl.pallas_call(
        _epilogue_kernel,
        grid=(nrow,),
        in_specs=[
            pl.BlockSpec((RB, VAL_DIM), lambda i: (i, 0)),
            pl.BlockSpec((RB, VAL_DIM), lambda i: (i, 0)),
            pl.BlockSpec((RB, DIM), lambda i: (i, 0)),
            pl.BlockSpec((1, VAL_DIM), lambda i: (0, 0)),
            pl.BlockSpec((1, DIM), lambda i: (0, 0)),
            pl.BlockSpec((VAL_DIM, DIM), lambda i: (0, 0)),
            pl.BlockSpec((DIM, E), lambda i: (0, 0)),
            pl.BlockSpec((DIM, SHARED_I), lambda i: (0, 0)),
            pl.BlockSpec((DIM, SHARED_I), lambda i: (0, 0)),
            pl.BlockSpec((SHARED_I, DIM), lambda i: (0, 0)),
            pl.BlockSpec((DIM, 1), lambda i: (0, 0)),
        ],
        out_specs=[
            pl.BlockSpec((RB, DIM), lambda i: (i, 0)),
            pl.BlockSpec((RB, DIM), lambda i: (i, 0)),
            pl.BlockSpec((RB, E), lambda i: (i, 0)),
        ],
        out_shape=[
            jax.ShapeDtypeStruct((S, DIM), f32),
            jax.ShapeDtypeStruct((S, DIM), f32),
            jax.ShapeDtypeStruct((S, E), f32),
        ],
    )(o, z, x2d, norm_weight[None, :], ffn_norm_weight[None, :],
      out_proj.astype(jnp.bfloat16), router_weight.T,
      shared_gate.astype(jnp.bfloat16), shared_up.astype(jnp.bfloat16),
      shared_down.astype(jnp.bfloat16), shared_expert_gate)

    x3 = pl.pallas_call(
        _expert_kernel,
        grid=(nrow, E),
        in_specs=[
            pl.BlockSpec((RB, DIM), lambda i, e: (i, 0)),
            pl.BlockSpec((1, 2 * MOE_I, DIM), lambda i, e: (e, 0, 0)),
            pl.BlockSpec((1, DIM, MOE_I), lambda i, e: (e, 0, 0)),
            pl.BlockSpec((RB, E), lambda i, e: (i, 0)),
            pl.BlockSpec((RB, DIM), lambda i, e: (i, 0)),
        ],
        out_specs=pl.BlockSpec((RB, DIM), lambda i, e: (i, 0)),
        out_shape=jax.ShapeDtypeStruct((S, DIM), f32),
        compiler_params=pltpu.CompilerParams(
            dimension_semantics=("parallel", "arbitrary")),
    )(h2, expert_gate_up.astype(jnp.bfloat16),
      expert_down.astype(jnp.bfloat16), cw, x2b)

    new_conv_state = qkv[S - (KCONV - 1):, :].T[None, :, :]
    return (x3[None, :, :], snew.reshape(1, NV, DK, DV), new_conv_state)


# speedup vs baseline: 15.0732x; 1.0265x over previous
import functools

import jax
import jax.numpy as jnp
from jax.experimental import pallas as pl
from jax.experimental.pallas import tpu as pltpu

B, S, DIM = 1, 2048, 1024
NK, DK, NV, DV = 4, 64, 8, 64
KEY_DIM, VAL_DIM = NK * DK, NV * DV
CONV_DIM, KCONV = 2 * KEY_DIM + VAL_DIM, 4
E, TOPK, MOE_I, SHARED_I = 8, 2, 512, 512
EPS = 1e-6
RB = 256
CHUNK = 64
NCHUNK = S // CHUNK


def _silu(x):
    return x * jax.nn.sigmoid(x)


def _rms(x, w1p):
    return x * jax.lax.rsqrt(jnp.mean(x * x, axis=-1, keepdims=True) + EPS) * w1p


def _prologue_kernel(x_ref, wall_ref, anw_ref, dtb_ref, alog_ref,
                     qkv_ref, z_ref, g_ref, beta_ref):
    h = _rms(x_ref[...], 1.0 + anw_ref[...])
    y = jnp.dot(h.astype(jnp.bfloat16), wall_ref[...],
                preferred_element_type=jnp.float32)
    qkv_ref[...] = y[:, :CONV_DIM]
    z_ref[...] = y[:, CONV_DIM:CONV_DIM + VAL_DIM]
    a = y[:, CONV_DIM + VAL_DIM:CONV_DIM + VAL_DIM + NV]
    b = y[:, CONV_DIM + VAL_DIM + NV:]
    g_ref[...] = -jnp.exp(alog_ref[...]) * jax.nn.softplus(a + dtb_ref[...])
    beta_ref[...] = jax.nn.sigmoid(b)


def _conv_kernel(xpad_ref, wt_ref, qn_ref, kn_ref, v_ref):
    acc = xpad_ref[5:5 + S, :] * wt_ref[0:1, :]
    for j in range(1, KCONV):
        acc = acc + xpad_ref[5 + j:5 + j + S, :] * wt_ref[j:j + 1, :]
    y = _silu(acc)
    for hh in range(NK):
        qh = y[:, hh * DK:(hh + 1) * DK]
        kh = y[:, KEY_DIM + hh * DK:KEY_DIM + (hh + 1) * DK]
        qn_ref[:, hh * DK:(hh + 1) * DK] = qh * jax.lax.rsqrt(
            jnp.sum(qh * qh, axis=-1, keepdims=True) + 1e-6)
        kn_ref[:, hh * DK:(hh + 1) * DK] = kh * jax.lax.rsqrt(
            jnp.sum(kh * kh, axis=-1, keepdims=True) + 1e-6)
    v_ref[...] = y[:, 2 * KEY_DIM:]


def _deltanet_kernel(qn_ref, kn_ref, v_ref, g_ref, beta_ref, s0_ref,
                     o_ref, sout_ref, s_scr):
    i = pl.program_id(0)

    @pl.when(i == 0)
    def _():
        s_scr[...] = s0_ref[...]

    row = jax.lax.broadcasted_iota(jnp.int32, (CHUNK, CHUNK), 0)
    col = jax.lax.broadcasted_iota(jnp.int32, (CHUNK, CHUNK), 1)
    incl = row >= col
    strict = row > col
    ltri = jnp.where(incl, 1.0, 0.0).astype(jnp.float32)
    eye = jnp.where(row == col, 1.0, 0.0).astype(jnp.float32)

    g_all = jnp.dot(ltri, g_ref[...], preferred_element_type=jnp.float32)

    for h in range(NV):
        hk = h // (NV // NK)
        q = qn_ref[:, hk * DK:(hk + 1) * DK]
        k = kn_ref[:, hk * DK:(hk + 1) * DK]
        v = v_ref[:, h * DV:(h + 1) * DV]
        gc = g_all[:, h:h + 1]
        beta = beta_ref[:, h:h + 1]
        s0 = s_scr[h * DK:(h + 1) * DK, :]

        dmat = jnp.exp(jnp.where(incl, gc - gc.T, -1e30))
        kkt = jnp.dot(k, k.T, preferred_element_type=jnp.float32)
        n = jnp.where(strict, -(beta * dmat * kkt), 0.0)
        p = eye + n
        q2 = jnp.dot(n, n, preferred_element_type=jnp.float32)
        for _ in range(5):
            p = p + jnp.dot(q2, p, preferred_element_type=jnp.float32)
            q2 = jnp.dot(q2, q2, preferred_element_type=jnp.float32)
        lam = jnp.exp(gc)
        w = (beta * lam) * k
        u = beta * v - jnp.dot(w, s0, preferred_element_type=jnp.float32)
        delta = jnp.dot(p, u, preferred_element_type=jnp.float32)

        qkt = jnp.dot(q, k.T, preferred_element_type=jnp.float32)
        o = jnp.dot(lam * q, s0, preferred_element_type=jnp.float32) + \
            jnp.dot(dmat * qkt, delta, preferred_element_type=jnp.float32)
        o_ref[:, h * DV:(h + 1) * DV] = o

        glast = gc[CHUNK - 1:CHUNK, :]
        kdec = jnp.exp(glast - gc) * k
        s1 = jnp.exp(glast) * s0 + jax.lax.dot_general(
            kdec, delta, (((0,), (0,)), ((), ())),
            preferred_element_type=jnp.float32)
        s_scr[h * DK:(h + 1) * DK, :] = s1

    @pl.when(i == NCHUNK - 1)
    def _():
        sout_ref[...] = s_scr[...]


def _epilogue_kernel(o_ref, z_ref, x_ref, nw_ref, fnw_ref, wout_ref,
                     rwt_ref, sg_ref, su_ref, sd_ref, seg_ref,
                     x2b_ref, h2_ref, cw_ref):
    o = _rms(o_ref[...], 1.0 + nw_ref[...]) * _silu(z_ref[...])
    attn = jnp.dot(o.astype(jnp.bfloat16), wout_ref[...],
                   preferred_element_type=jnp.float32)
    x2 = x_ref[...] + attn
    h2 = _rms(x2, 1.0 + fnw_ref[...])
    h2b = h2.astype(jnp.bfloat16)
    h2_ref[...] = h2b

    logits = jnp.dot(h2, rwt_ref[...], preferred_element_type=jnp.float32)
    m = jnp.max(logits, axis=-1, keepdims=True)
    ex = jnp.exp(logits - m)
    probs = ex / jnp.sum(ex, axis=-1, keepdims=True)

    lane = jax.lax.broadcasted_iota(jnp.int32, probs.shape, 1)
    m0 = jnp.max(probs, axis=-1, keepdims=True)
    i0 = jnp.min(jnp.where(probs == m0, lane, E), axis=-1, keepdims=True)
    sel0 = lane == i0
    p2 = jnp.where(sel0, -jnp.inf, probs)
    m1 = jnp.max(p2, axis=-1, keepdims=True)
    i1 = jnp.min(jnp.where(p2 == m1, lane, E), axis=-1, keepdims=True)
    sel1 = lane == i1
    denom = m0 + m1
    cw_ref[...] = (jnp.where(sel0, m0, 0.0) + jnp.where(sel1, m1, 0.0)) / denom

    sact = _silu(jnp.dot(h2b, sg_ref[...], preferred_element_type=jnp.float32)) * \
        jnp.dot(h2b, su_ref[...], preferred_element_type=jnp.float32)
    shared = jnp.dot(sact.astype(jnp.bfloat16), sd_ref[...],
                     preferred_element_type=jnp.float32)
    gate = jax.nn.sigmoid(
        jnp.dot(h2, seg_ref[...], preferred_element_type=jnp.float32))
    x2b_ref[...] = x2 + gate * shared


def _expert_kernel(h2_ref, egu_ref, ed_ref, cw_ref, x2b_ref, out_ref,
                   acc_scr):
    e = pl.program_id(0)
    for rb in range(S // RB):
        sl = pl.ds(rb * RB, RB)
        gu = jax.lax.dot_general(h2_ref[sl, :], egu_ref[0],
                                 (((1,), (1,)), ((), ())),
                                 preferred_element_type=jnp.float32)
        act = _silu(gu[:, :MOE_I]) * gu[:, MOE_I:]
        down = jax.lax.dot_general(act.astype(jnp.bfloat16), ed_ref[0],
                                   (((1,), (1,)), ((), ())),
                                   preferred_element_type=jnp.float32)
        lane = jax.lax.broadcasted_iota(jnp.int32, (RB, E), 1)
        we = jnp.sum(jnp.where(lane == e, cw_ref[sl, :], 0.0),
                     axis=-1, keepdims=True)
        contrib = we * down

        @pl.when(e == 0)
        def _():
            acc_scr[sl, :] = contrib

        @pl.when(e > 0)
        def _():
            acc_scr[sl, :] = acc_scr[sl, :] + contrib

    @pl.when(e == E - 1)
    def _():
        out_ref[...] = x2b_ref[...] + acc_scr[...]


def kernel(x, state, conv_state, attention_norm_weight, ffn_norm_weight,
           in_proj_qkv, in_proj_z, in_proj_a, in_proj_b, conv1d_weight,
           dt_bias, A_log, norm_weight, out_proj, router_weight,
           expert_gate_up, expert_down, shared_gate, shared_up, shared_down,
           shared_expert_gate, linear_layer_idx):
    f32 = jnp.float32
    x2d = x[0]
    w_all = jnp.concatenate([in_proj_qkv, in_proj_z, in_proj_a, in_proj_b],
                            axis=1).astype(jnp.bfloat16)
    anw = attention_norm_weight[None, :]
    dtb = dt_bias[None, :]
    alog = A_log[None, :]

    nrow = S // RB
    qkv, z, g, beta = pl.pallas_call(
        _prologue_kernel,
        grid=(nrow,),
        in_specs=[
            pl.BlockSpec((RB, DIM), lambda i: (i, 0)),
            pl.BlockSpec((DIM, w_all.shape[1]), lambda i: (0, 0)),
            pl.BlockSpec((1, DIM), lambda i: (0, 0)),
            pl.BlockSpec((1, NV), lambda i: (0, 0)),
            pl.BlockSpec((1, NV), lambda i: (0, 0)),
        ],
        out_specs=[
            pl.BlockSpec((RB, CONV_DIM), lambda i: (i, 0)),
            pl.BlockSpec((RB, VAL_DIM), lambda i: (i, 0)),
            pl.BlockSpec((RB, NV), lambda i: (i, 0)),
            pl.BlockSpec((RB, NV), lambda i: (i, 0)),
        ],
        out_shape=[
            jax.ShapeDtypeStruct((S, CONV_DIM), f32),
            jax.ShapeDtypeStruct((S, VAL_DIM), f32),
            jax.ShapeDtypeStruct((S, NV), f32),
            jax.ShapeDtypeStruct((S, NV), f32),
        ],
    )(x2d, w_all, anw, dtb, alog)

    pre = jnp.zeros((8, CONV_DIM), f32).at[5:8, :].set(conv_state[0].T)
    xpad = jnp.concatenate([pre, qkv], axis=0)
    wt = conv1d_weight.T

    qn, kn, v = pl.pallas_call(
        _conv_kernel,
        in_specs=[
            pl.BlockSpec((S + 8, CONV_DIM), lambda: (0, 0)),
            pl.BlockSpec((KCONV, CONV_DIM), lambda: (0, 0)),
        ],
        out_specs=[
            pl.BlockSpec((S, KEY_DIM), lambda: (0, 0)),
            pl.BlockSpec((S, KEY_DIM), lambda: (0, 0)),
            pl.BlockSpec((S, VAL_DIM), lambda: (0, 0)),
        ],
        out_shape=[
            jax.ShapeDtypeStruct((S, KEY_DIM), f32),
            jax.ShapeDtypeStruct((S, KEY_DIM), f32),
            jax.ShapeDtypeStruct((S, VAL_DIM), f32),
        ],
    )(xpad, wt)

    s0 = state[0].reshape(NV * DK, DV)
    o, snew = pl.pallas_call(
        _deltanet_kernel,
        grid=(NCHUNK,),
        in_specs=[
            pl.BlockSpec((CHUNK, KEY_DIM), lambda i: (i, 0)),
            pl.BlockSpec((CHUNK, KEY_DIM), lambda i: (i, 0)),
            pl.BlockSpec((CHUNK, VAL_DIM), lambda i: (i, 0)),
            pl.BlockSpec((CHUNK, NV), lambda i: (i, 0)),
            pl.BlockSpec((CHUNK, NV), lambda i: (i, 0)),
            pl.BlockSpec((NV * DK, DV), lambda i: (0, 0)),
        ],
        out_specs=[
            pl.BlockSpec((CHUNK, VAL_DIM), lambda i: (i, 0)),
            pl.BlockSpec((NV * DK, DV), lambda i: (0, 0)),
        ],
        out_shape=[
            jax.ShapeDtypeStruct((S, VAL_DIM), f32),
            jax.ShapeDtypeStruct((NV * DK, DV), f32),
        ],
        scratch_shapes=[pltpu.VMEM((NV * DK, DV), f32)],
        compiler_params=pltpu.CompilerParams(
            dimension_semantics=("arbitrary",)),
    )(qn, kn, v, g, beta, s0)

    x2b, h2, cw = pl.pallas_call(
        _epilogue_kernel,
        grid=(nrow,),
        in_specs=[
            pl.BlockSpec((RB, VAL_DIM), lambda i: (i, 0)),
            pl.BlockSpec((RB, VAL_DIM), lambda i: (i, 0)),
            pl.BlockSpec((RB, DIM), lambda i: (i, 0)),
            pl.BlockSpec((1, VAL_DIM), lambda i: (0, 0)),
            pl.BlockSpec((1, DIM), lambda i: (0, 0)),
            pl.BlockSpec((VAL_DIM, DIM), lambda i: (0, 0)),
            pl.BlockSpec((DIM, E), lambda i: (0, 0)),
            pl.BlockSpec((DIM, SHARED_I), lambda i: (0, 0)),
            pl.BlockSpec((DIM, SHARED_I), lambda i: (0, 0)),
            pl.BlockSpec((SHARED_I, DIM), lambda i: (0, 0)),
            pl.BlockSpec((DIM, 1), lambda i: (0, 0)),
        ],
        out_specs=[
            pl.BlockSpec((RB, DIM), lambda i: (i, 0)),
            pl.BlockSpec((RB, DIM), lambda i: (i, 0)),
            pl.BlockSpec((RB, E), lambda i: (i, 0)),
        ],
        out_shape=[
            jax.ShapeDtypeStruct((S, DIM), f32),
            jax.ShapeDtypeStruct((S, DIM), jnp.bfloat16),
            jax.ShapeDtypeStruct((S, E), f32),
        ],
    )(o, z, x2d, norm_weight[None, :], ffn_norm_weight[None, :],
      out_proj.astype(jnp.bfloat16), router_weight.T,
      shared_gate.astype(jnp.bfloat16), shared_up.astype(jnp.bfloat16),
      shared_down.astype(jnp.bfloat16), shared_expert_gate)

    x3 = pl.pallas_call(
        _expert_kernel,
        grid=(E,),
        in_specs=[
            pl.BlockSpec((S, DIM), lambda e: (0, 0)),
            pl.BlockSpec((1, 2 * MOE_I, DIM), lambda e: (e, 0, 0)),
            pl.BlockSpec((1, DIM, MOE_I), lambda e: (e, 0, 0)),
            pl.BlockSpec((S, E), lambda e: (0, 0)),
            pl.BlockSpec((S, DIM), lambda e: (0, 0)),
        ],
        out_specs=pl.BlockSpec((S, DIM), lambda e: (0, 0)),
        out_shape=jax.ShapeDtypeStruct((S, DIM), f32),
        scratch_shapes=[pltpu.VMEM((S, DIM), f32)],
        compiler_params=pltpu.CompilerParams(
            dimension_semantics=("arbitrary",)),
    )(h2, expert_gate_up.astype(jnp.bfloat16),
      expert_down.astype(jnp.bfloat16), cw, x2b)

    new_conv_state = qkv[S - (KCONV - 1):, :].T[None, :, :]
    return (x3[None, :, :], snew.reshape(1, NV, DK, DV), new_conv_state)


# speedup vs baseline: 25.3039x; 1.6787x over previous
import functools

import jax
import jax.numpy as jnp
from jax.experimental import pallas as pl
from jax.experimental.pallas import tpu as pltpu

B, S, DIM = 1, 2048, 1024
NK, DK, NV, DV = 4, 64, 8, 64
KEY_DIM, VAL_DIM = NK * DK, NV * DV
CONV_DIM, KCONV = 2 * KEY_DIM + VAL_DIM, 4
E, TOPK, MOE_I, SHARED_I = 8, 2, 512, 512
EPS = 1e-6
RB = 256
CHUNK = 64
NCHUNK = S // CHUNK


def _silu(x):
    return x * jax.nn.sigmoid(x)


def _rms(x, w1p):
    return x * jax.lax.rsqrt(jnp.mean(x * x, axis=-1, keepdims=True) + EPS) * w1p


def _prologue_kernel(x_ref, wall_ref, anw_ref, dtb_ref, alog_ref,
                     qkv_ref, z_ref, g_ref, beta_ref):
    h = _rms(x_ref[...], 1.0 + anw_ref[...])
    y = jnp.dot(h.astype(jnp.bfloat16), wall_ref[...],
                preferred_element_type=jnp.float32)
    qkv_ref[...] = y[:, :CONV_DIM]
    z_ref[...] = y[:, CONV_DIM:CONV_DIM + VAL_DIM]
    a = y[:, CONV_DIM + VAL_DIM:CONV_DIM + VAL_DIM + NV]
    b = y[:, CONV_DIM + VAL_DIM + NV:]
    g_ref[...] = -jnp.exp(alog_ref[...]) * jax.nn.softplus(a + dtb_ref[...])
    beta_ref[...] = jax.nn.sigmoid(b)


def _conv_kernel(xpad_ref, wt_ref, qn_ref, kn_ref, v_ref):
    acc = xpad_ref[5:5 + S, :] * wt_ref[0:1, :]
    for j in range(1, KCONV):
        acc = acc + xpad_ref[5 + j:5 + j + S, :] * wt_ref[j:j + 1, :]
    y = _silu(acc)
    for hh in range(NK):
        qh = y[:, hh * DK:(hh + 1) * DK]
        kh = y[:, KEY_DIM + hh * DK:KEY_DIM + (hh + 1) * DK]
        qn_ref[:, hh * DK:(hh + 1) * DK] = qh * jax.lax.rsqrt(
            jnp.sum(qh * qh, axis=-1, keepdims=True) + 1e-6)
        kn_ref[:, hh * DK:(hh + 1) * DK] = kh * jax.lax.rsqrt(
            jnp.sum(kh * kh, axis=-1, keepdims=True) + 1e-6)
    v_ref[...] = y[:, 2 * KEY_DIM:]


_CT = (((1,), (1,)), ((), ()))
_NN = (((1,), (0,)), ((), ()))
_TN = (((0,), (0,)), ((), ()))


def _mm(a, b, dims):
    return jax.lax.dot_general(a.astype(jnp.bfloat16), b.astype(jnp.bfloat16),
                               dims, preferred_element_type=jnp.float32)


def _deltanet_kernel(qn_ref, kn_ref, v_ref, g_ref, beta_ref, s0_ref,
                     o_ref, sout_ref, s_scr):
    i = pl.program_id(0)

    @pl.when(i == 0)
    def _():
        s_scr[...] = s0_ref[...]

    s_all = s_scr[...]
    row = jax.lax.broadcasted_iota(jnp.int32, (CHUNK, CHUNK), 0)
    col = jax.lax.broadcasted_iota(jnp.int32, (CHUNK, CHUNK), 1)
    incl = row >= col
    ltri = jnp.where(incl, 1.0, 0.0).astype(jnp.float32)
    eye = jnp.where(row == col, 1.0, 0.0).astype(jnp.float32)

    g_all = jnp.dot(ltri, g_ref[...], preferred_element_type=jnp.float32)
    rep = NV // NK

    ks = [kn_ref[:, hk * DK:(hk + 1) * DK] for hk in range(NK)]
    qs = [qn_ref[:, hk * DK:(hk + 1) * DK] for hk in range(NK)]
    kkts = [_mm(k, k, _CT) for k in ks]
    qkts = [_mm(q, k, _CT) for q, k in zip(qs, ks)]

    gc = [g_all[:, h:h + 1] for h in range(NV)]
    beta = [beta_ref[:, h:h + 1] for h in range(NV)]
    dmat = [jnp.exp(jnp.where(incl, gc[h] - gc[h].T, -1e30))
            for h in range(NV)]
    lam = [jnp.exp(gc[h]) for h in range(NV)]
    glast = [gc[h][CHUNK - 1:CHUNK, :] for h in range(NV)]

    n = [jnp.where(row > col, -(beta[h] * dmat[h] * kkts[h // rep]), 0.0)
         for h in range(NV)]
    p = [eye + n[h] for h in range(NV)]
    q2 = [_mm(n[h], n[h], _NN) for h in range(NV)]
    for lvl in range(5):
        p = [p[h] + _mm(q2[h], p[h], _NN) for h in range(NV)]
        if lvl < 4:
            q2 = [_mm(q2[h], q2[h], _NN) for h in range(NV)]

    s0 = [s_all[h * DK:(h + 1) * DK, :] for h in range(NV)]
    vs = [v_ref[:, h * DV:(h + 1) * DV] for h in range(NV)]
    w = [(beta[h] * lam[h]) * ks[h // rep] for h in range(NV)]
    u = [beta[h] * vs[h] - _mm(w[h], s0[h], _NN) for h in range(NV)]
    delta = [_mm(p[h], u[h], _NN) for h in range(NV)]

    o = [_mm(lam[h] * qs[h // rep], s0[h], _NN) +
         _mm(dmat[h] * qkts[h // rep], delta[h], _NN) for h in range(NV)]
    o_ref[...] = jnp.concatenate(o, axis=1)

    kdec = [jnp.exp(glast[h] - gc[h]) * ks[h // rep] for h in range(NV)]
    s1 = [jnp.exp(glast[h]) * s0[h] + _mm(kdec[h], delta[h], _TN)
          for h in range(NV)]
    s_new = jnp.concatenate(s1, axis=0)
    s_scr[...] = s_new

    @pl.when(i == NCHUNK - 1)
    def _():
        sout_ref[...] = s_new


def _epilogue_kernel(o_ref, z_ref, x_ref, nw_ref, fnw_ref, wout_ref,
                     rwt_ref, sg_ref, su_ref, sd_ref, seg_ref,
                     x2b_ref, h2_ref, cw_ref):
    o = _rms(o_ref[...], 1.0 + nw_ref[...]) * _silu(z_ref[...])
    attn = jnp.dot(o.astype(jnp.bfloat16), wout_ref[...],
                   preferred_element_type=jnp.float32)
    x2 = x_ref[...] + attn
    h2 = _rms(x2, 1.0 + fnw_ref[...])
    h2b = h2.astype(jnp.bfloat16)
    h2_ref[...] = h2b

    logits = jnp.dot(h2, rwt_ref[...], preferred_element_type=jnp.float32)
    m = jnp.max(logits, axis=-1, keepdims=True)
    ex = jnp.exp(logits - m)
    probs = ex / jnp.sum(ex, axis=-1, keepdims=True)

    lane = jax.lax.broadcasted_iota(jnp.int32, probs.shape, 1)
    m0 = jnp.max(probs, axis=-1, keepdims=True)
    i0 = jnp.min(jnp.where(probs == m0, lane, E), axis=-1, keepdims=True)
    sel0 = lane == i0
    p2 = jnp.where(sel0, -jnp.inf, probs)
    m1 = jnp.max(p2, axis=-1, keepdims=True)
    i1 = jnp.min(jnp.where(p2 == m1, lane, E), axis=-1, keepdims=True)
    sel1 = lane == i1
    denom = m0 + m1
    cw_ref[...] = (jnp.where(sel0, m0, 0.0) + jnp.where(sel1, m1, 0.0)) / denom

    sact = _silu(jnp.dot(h2b, sg_ref[...], preferred_element_type=jnp.float32)) * \
        jnp.dot(h2b, su_ref[...], preferred_element_type=jnp.float32)
    shared = jnp.dot(sact.astype(jnp.bfloat16), sd_ref[...],
                     preferred_element_type=jnp.float32)
    gate = jax.nn.sigmoid(
        jnp.dot(h2, seg_ref[...], preferred_element_type=jnp.float32))
    x2b_ref[...] = x2 + gate * shared


def _expert_kernel(h2_ref, egu_ref, ed_ref, cw_ref, x2b_ref, out_ref,
                   acc_scr):
    e = pl.program_id(0)
    for rb in range(S // RB):
        sl = pl.ds(rb * RB, RB)
        gu = jax.lax.dot_general(h2_ref[sl, :], egu_ref[0],
                                 (((1,), (1,)), ((), ())),
                                 preferred_element_type=jnp.float32)
        act = _silu(gu[:, :MOE_I]) * gu[:, MOE_I:]
        down = jax.lax.dot_general(act.astype(jnp.bfloat16), ed_ref[0],
                                   (((1,), (1,)), ((), ())),
                                   preferred_element_type=jnp.float32)
        lane = jax.lax.broadcasted_iota(jnp.int32, (RB, E), 1)
        we = jnp.sum(jnp.where(lane == e, cw_ref[sl, :], 0.0),
                     axis=-1, keepdims=True)
        contrib = we * down

        @pl.when(e == 0)
        def _():
            acc_scr[sl, :] = contrib

        @pl.when(e > 0)
        def _():
            acc_scr[sl, :] = acc_scr[sl, :] + contrib

    @pl.when(e == E - 1)
    def _():
        out_ref[...] = x2b_ref[...] + acc_scr[...]


def kernel(x, state, conv_state, attention_norm_weight, ffn_norm_weight,
           in_proj_qkv, in_proj_z, in_proj_a, in_proj_b, conv1d_weight,
           dt_bias, A_log, norm_weight, out_proj, router_weight,
           expert_gate_up, expert_down, shared_gate, shared_up, shared_down,
           shared_expert_gate, linear_layer_idx):
    f32 = jnp.float32
    x2d = x[0]
    w_all = jnp.concatenate([in_proj_qkv, in_proj_z, in_proj_a, in_proj_b],
                            axis=1).astype(jnp.bfloat16)
    anw = attention_norm_weight[None, :]
    dtb = dt_bias[None, :]
    alog = A_log[None, :]

    nrow = S // RB
    qkv, z, g, beta = pl.pallas_call(
        _prologue_kernel,
        grid=(nrow,),
        in_specs=[
            pl.BlockSpec((RB, DIM), lambda i: (i, 0)),
            pl.BlockSpec((DIM, w_all.shape[1]), lambda i: (0, 0)),
            pl.BlockSpec((1, DIM), lambda i: (0, 0)),
            pl.BlockSpec((1, NV), lambda i: (0, 0)),
            pl.BlockSpec((1, NV), lambda i: (0, 0)),
        ],
        out_specs=[
            pl.BlockSpec((RB, CONV_DIM), lambda i: (i, 0)),
            pl.BlockSpec((RB, VAL_DIM), lambda i: (i, 0)),
            pl.BlockSpec((RB, NV), lambda i: (i, 0)),
            pl.BlockSpec((RB, NV), lambda i: (i, 0)),
        ],
        out_shape=[
            jax.ShapeDtypeStruct((S, CONV_DIM), f32),
            jax.ShapeDtypeStruct((S, VAL_DIM), f32),
            jax.ShapeDtypeStruct((S, NV), f32),
            jax.ShapeDtypeStruct((S, NV), f32),
        ],
    )(x2d, w_all, anw, dtb, alog)

    pre = jnp.zeros((8, CONV_DIM), f32).at[5:8, :].set(conv_state[0].T)
    xpad = jnp.concatenate([pre, qkv], axis=0)
    wt = conv1d_weight.T

    qn, kn, v = pl.pallas_call(
        _conv_kernel,
        in_specs=[
            pl.BlockSpec((S + 8, CONV_DIM), lambda: (0, 0)),
            pl.BlockSpec((KCONV, CONV_DIM), lambda: (0, 0)),
        ],
        out_specs=[
            pl.BlockSpec((S, KEY_DIM), lambda: (0, 0)),
            pl.BlockSpec((S, KEY_DIM), lambda: (0, 0)),
            pl.BlockSpec((S, VAL_DIM), lambda: (0, 0)),
        ],
        out_shape=[
            jax.ShapeDtypeStruct((S, KEY_DIM), f32),
            jax.ShapeDtypeStruct((S, KEY_DIM), f32),
            jax.ShapeDtypeStruct((S, VAL_DIM), f32),
        ],
    )(xpad, wt)

    s0 = state[0].reshape(NV * DK, DV)
    o, snew = pl.pallas_call(
        _deltanet_kernel,
        grid=(NCHUNK,),
        in_specs=[
            pl.BlockSpec((CHUNK, KEY_DIM), lambda i: (i, 0)),
            pl.BlockSpec((CHUNK, KEY_DIM), lambda i: (i, 0)),
            pl.BlockSpec((CHUNK, VAL_DIM), lambda i: (i, 0)),
            pl.BlockSpec((CHUNK, NV), lambda i: (i, 0)),
            pl.BlockSpec((CHUNK, NV), lambda i: (i, 0)),
            pl.BlockSpec((NV * DK, DV), lambda i: (0, 0)),
        ],
        out_specs=[
            pl.BlockSpec((CHUNK, VAL_DIM), lambda i: (i, 0)),
            pl.BlockSpec((NV * DK, DV), lambda i: (0, 0)),
        ],
        out_shape=[
            jax.ShapeDtypeStruct((S, VAL_DIM), f32),
            jax.ShapeDtypeStruct((NV * DK, DV), f32),
        ],
        scratch_shapes=[pltpu.VMEM((NV * DK, DV), f32)],
        compiler_params=pltpu.CompilerParams(
            dimension_semantics=("arbitrary",)),
    )(qn, kn, v, g, beta, s0)

    x2b, h2, cw = pl.pallas_call(
        _epilogue_kernel,
        grid=(nrow,),
        in_specs=[
            pl.BlockSpec((RB, VAL_DIM), lambda i: (i, 0)),
            pl.BlockSpec((RB, VAL_DIM), lambda i: (i, 0)),
            pl.BlockSpec((RB, DIM), lambda i: (i, 0)),
            pl.BlockSpec((1, VAL_DIM), lambda i: (0, 0)),
            pl.BlockSpec((1, DIM), lambda i: (0, 0)),
            pl.BlockSpec((VAL_DIM, DIM), lambda i: (0, 0)),
            pl.BlockSpec((DIM, E), lambda i: (0, 0)),
            pl.BlockSpec((DIM, SHARED_I), lambda i: (0, 0)),
            pl.BlockSpec((DIM, SHARED_I), lambda i: (0, 0)),
            pl.BlockSpec((SHARED_I, DIM), lambda i: (0, 0)),
            pl.BlockSpec((DIM, 1), lambda i: (0, 0)),
        ],
        out_specs=[
            pl.BlockSpec((RB, DIM), lambda i: (i, 0)),
            pl.BlockSpec((RB, DIM), lambda i: (i, 0)),
            pl.BlockSpec((RB, E), lambda i: (i, 0)),
        ],
        out_shape=[
            jax.ShapeDtypeStruct((S, DIM), f32),
            jax.ShapeDtypeStruct((S, DIM), jnp.bfloat16),
            jax.ShapeDtypeStruct((S, E), f32),
        ],
    )(o, z, x2d, norm_weight[None, :], ffn_norm_weight[None, :],
      out_proj.astype(jnp.bfloat16), router_weight.T,
      shared_gate.astype(jnp.bfloat16), shared_up.astype(jnp.bfloat16),
      shared_down.astype(jnp.bfloat16), shared_expert_gate)

    x3 = pl.pallas_call(
        _expert_kernel,
        grid=(E,),
        in_specs=[
            pl.BlockSpec((S, DIM), lambda e: (0, 0)),
            pl.BlockSpec((1, 2 * MOE_I, DIM), lambda e: (e, 0, 0)),
            pl.BlockSpec((1, DIM, MOE_I), lambda e: (e, 0, 0)),
            pl.BlockSpec((S, E), lambda e: (0, 0)),
            pl.BlockSpec((S, DIM), lambda e: (0, 0)),
        ],
        out_specs=pl.BlockSpec((S, DIM), lambda e: (0, 0)),
        out_shape=jax.ShapeDtypeStruct((S, DIM), f32),
        scratch_shapes=[pltpu.VMEM((S, DIM), f32)],
        compiler_params=pltpu.CompilerParams(
            dimension_semantics=("arbitrary",)),
    )(h2, expert_gate_up.astype(jnp.bfloat16),
      expert_down.astype(jnp.bfloat16), cw, x2b)

    new_conv_state = qkv[S - (KCONV - 1):, :].T[None, :, :]
    return (x3[None, :, :], snew.reshape(1, NV, DK, DV), new_conv_state)


# speedup vs baseline: 30.1878x; 1.1930x over previous
import functools

import jax
import jax.numpy as jnp
from jax.experimental import pallas as pl
from jax.experimental.pallas import tpu as pltpu

B, S, DIM = 1, 2048, 1024
NK, DK, NV, DV = 4, 64, 8, 64
KEY_DIM, VAL_DIM = NK * DK, NV * DV
CONV_DIM, KCONV = 2 * KEY_DIM + VAL_DIM, 4
E, TOPK, MOE_I, SHARED_I = 8, 2, 512, 512
EPS = 1e-6
RB = 256
CHUNK = 64
NCHUNK = S // CHUNK


def _silu(x):
    return x * jax.nn.sigmoid(x)


def _rms(x, w1p):
    return x * jax.lax.rsqrt(jnp.mean(x * x, axis=-1, keepdims=True) + EPS) * w1p


def _prologue_kernel(x_ref, wqkv_ref, wz_ref, wa_ref, wb_ref, anw_ref,
                     dtb_ref, alog_ref, qkv_ref, z_ref, g_ref, beta_ref):
    h = _rms(x_ref[...], 1.0 + anw_ref[...])
    hb = h.astype(jnp.bfloat16)
    qkv_ref[...] = jnp.dot(hb, wqkv_ref[...].astype(jnp.bfloat16),
                           preferred_element_type=jnp.float32)
    z_ref[...] = jnp.dot(hb, wz_ref[...].astype(jnp.bfloat16),
                         preferred_element_type=jnp.float32)
    a = jnp.dot(h, wa_ref[...], preferred_element_type=jnp.float32)
    b = jnp.dot(h, wb_ref[...], preferred_element_type=jnp.float32)
    g_ref[...] = -jnp.exp(alog_ref[...]) * jax.nn.softplus(a + dtb_ref[...])
    beta_ref[...] = jax.nn.sigmoid(b)


def _conv_kernel(qkv_ref, cs_ref, wt_ref, qn_ref, kn_ref, v_ref, xpad_scr):
    xpad_scr[5:8, :] = cs_ref[...]
    xpad_scr[8:, :] = qkv_ref[...]
    acc = xpad_scr[5:5 + S, :] * wt_ref[0:1, :]
    for j in range(1, KCONV):
        acc = acc + xpad_scr[5 + j:5 + j + S, :] * wt_ref[j:j + 1, :]
    y = _silu(acc)
    for hh in range(NK):
        qh = y[:, hh * DK:(hh + 1) * DK]
        kh = y[:, KEY_DIM + hh * DK:KEY_DIM + (hh + 1) * DK]
        qn_ref[:, hh * DK:(hh + 1) * DK] = qh * jax.lax.rsqrt(
            jnp.sum(qh * qh, axis=-1, keepdims=True) + 1e-6)
        kn_ref[:, hh * DK:(hh + 1) * DK] = kh * jax.lax.rsqrt(
            jnp.sum(kh * kh, axis=-1, keepdims=True) + 1e-6)
    v_ref[...] = y[:, 2 * KEY_DIM:]


_CT = (((1,), (1,)), ((), ()))
_NN = (((1,), (0,)), ((), ()))
_TN = (((0,), (0,)), ((), ()))


def _mm(a, b, dims):
    return jax.lax.dot_general(a.astype(jnp.bfloat16), b.astype(jnp.bfloat16),
                               dims, preferred_element_type=jnp.float32)


def _deltanet_kernel(qn_ref, kn_ref, v_ref, g_ref, beta_ref, s0_ref,
                     o_ref, sout_ref, s_scr):
    i = pl.program_id(0)

    @pl.when(i == 0)
    def _():
        s_scr[...] = s0_ref[...]

    s_all = s_scr[...]
    row = jax.lax.broadcasted_iota(jnp.int32, (CHUNK, CHUNK), 0)
    col = jax.lax.broadcasted_iota(jnp.int32, (CHUNK, CHUNK), 1)
    incl = row >= col
    ltri = jnp.where(incl, 1.0, 0.0).astype(jnp.float32)
    eye = jnp.where(row == col, 1.0, 0.0).astype(jnp.float32)

    g_all = jnp.dot(ltri, g_ref[...], preferred_element_type=jnp.float32)
    rep = NV // NK

    ks = [kn_ref[:, hk * DK:(hk + 1) * DK] for hk in range(NK)]
    qs = [qn_ref[:, hk * DK:(hk + 1) * DK] for hk in range(NK)]
    kkts = [_mm(k, k, _CT) for k in ks]
    qkts = [_mm(q, k, _CT) for q, k in zip(qs, ks)]

    gc = [g_all[:, h:h + 1] for h in range(NV)]
    beta = [beta_ref[:, h:h + 1] for h in range(NV)]
    dmat = [jnp.exp(jnp.where(incl, gc[h] - gc[h].T, -1e30))
            for h in range(NV)]
    lam = [jnp.exp(gc[h]) for h in range(NV)]
    glast = [gc[h][CHUNK - 1:CHUNK, :] for h in range(NV)]

    n = [jnp.where(row > col, -(beta[h] * dmat[h] * kkts[h // rep]), 0.0)
         for h in range(NV)]
    p = [eye + n[h] for h in range(NV)]
    q2 = [_mm(n[h], n[h], _NN) for h in range(NV)]
    for lvl in range(5):
        p = [p[h] + _mm(q2[h], p[h], _NN) for h in range(NV)]
        if lvl < 4:
            q2 = [_mm(q2[h], q2[h], _NN) for h in range(NV)]

    s0 = [s_all[h * DK:(h + 1) * DK, :] for h in range(NV)]
    vs = [v_ref[:, h * DV:(h + 1) * DV] for h in range(NV)]
    w = [(beta[h] * lam[h]) * ks[h // rep] for h in range(NV)]
    u = [beta[h] * vs[h] - _mm(w[h], s0[h], _NN) for h in range(NV)]
    delta = [_mm(p[h], u[h], _NN) for h in range(NV)]

    o = [_mm(lam[h] * qs[h // rep], s0[h], _NN) +
         _mm(dmat[h] * qkts[h // rep], delta[h], _NN) for h in range(NV)]
    o_ref[...] = jnp.concatenate(o, axis=1)

    kdec = [jnp.exp(glast[h] - gc[h]) * ks[h // rep] for h in range(NV)]
    s1 = [jnp.exp(glast[h]) * s0[h] + _mm(kdec[h], delta[h], _TN)
          for h in range(NV)]
    s_new = jnp.concatenate(s1, axis=0)
    s_scr[...] = s_new

    @pl.when(i == NCHUNK - 1)
    def _():
        sout_ref[...] = s_new


def _epilogue_kernel(o_ref, z_ref, x_ref, nw_ref, fnw_ref, wout_ref,
                     rwt_ref, sg_ref, su_ref, sd_ref, seg_ref,
                     x2b_ref, h2_ref, cw_ref):
    o = _rms(o_ref[...], 1.0 + nw_ref[...]) * _silu(z_ref[...])
    attn = jnp.dot(o.astype(jnp.bfloat16), wout_ref[...].astype(jnp.bfloat16),
                   preferred_element_type=jnp.float32)
    x2 = x_ref[...] + attn
    h2 = _rms(x2, 1.0 + fnw_ref[...])
    h2b = h2.astype(jnp.bfloat16)
    h2_ref[...] = h2b

    logits = jnp.dot(h2, rwt_ref[...], preferred_element_type=jnp.float32)
    m = jnp.max(logits, axis=-1, keepdims=True)
    ex = jnp.exp(logits - m)
    probs = ex / jnp.sum(ex, axis=-1, keepdims=True)

    lane = jax.lax.broadcasted_iota(jnp.int32, probs.shape, 1)
    m0 = jnp.max(probs, axis=-1, keepdims=True)
    i0 = jnp.min(jnp.where(probs == m0, lane, E), axis=-1, keepdims=True)
    sel0 = lane == i0
    p2 = jnp.where(sel0, -jnp.inf, probs)
    m1 = jnp.max(p2, axis=-1, keepdims=True)
    i1 = jnp.min(jnp.where(p2 == m1, lane, E), axis=-1, keepdims=True)
    sel1 = lane == i1
    denom = m0 + m1
    cw_ref[...] = (jnp.where(sel0, m0, 0.0) + jnp.where(sel1, m1, 0.0)) / denom

    sact = _silu(jnp.dot(h2b, sg_ref[...].astype(jnp.bfloat16),
                         preferred_element_type=jnp.float32)) * \
        jnp.dot(h2b, su_ref[...].astype(jnp.bfloat16),
                preferred_element_type=jnp.float32)
    shared = jnp.dot(sact.astype(jnp.bfloat16),
                     sd_ref[...].astype(jnp.bfloat16),
                     preferred_element_type=jnp.float32)
    gate = jax.nn.sigmoid(
        jnp.dot(h2, seg_ref[...], preferred_element_type=jnp.float32))
    x2b_ref[...] = x2 + gate * shared


def _expert_kernel(h2_ref, egu_ref, ed_ref, cw_ref, x2b_ref, out_ref,
                   acc_scr):
    e = pl.program_id(0)
    egub = egu_ref[0].astype(jnp.bfloat16)
    edb = ed_ref[0].astype(jnp.bfloat16)
    for rb in range(S // RB):
        sl = pl.ds(rb * RB, RB)
        gu = jax.lax.dot_general(h2_ref[sl, :], egub,
                                 (((1,), (1,)), ((), ())),
                                 preferred_element_type=jnp.float32)
        act = _silu(gu[:, :MOE_I]) * gu[:, MOE_I:]
        down = jax.lax.dot_general(act.astype(jnp.bfloat16), edb,
                                   (((1,), (1,)), ((), ())),
                                   preferred_element_type=jnp.float32)
        lane = jax.lax.broadcasted_iota(jnp.int32, (RB, E), 1)
        we = jnp.sum(jnp.where(lane == e, cw_ref[sl, :], 0.0),
                     axis=-1, keepdims=True)
        contrib = we * down

        @pl.when(e == 0)
        def _():
            acc_scr[sl, :] = contrib

        @pl.when(e > 0)
        def _():
            acc_scr[sl, :] = acc_scr[sl, :] + contrib

    @pl.when(e == E - 1)
    def _():
        out_ref[...] = x2b_ref[...] + acc_scr[...]


def kernel(x, state, conv_state, attention_norm_weight, ffn_norm_weight,
           in_proj_qkv, in_proj_z, in_proj_a, in_proj_b, conv1d_weight,
           dt_bias, A_log, norm_weight, out_proj, router_weight,
           expert_gate_up, expert_down, shared_gate, shared_up, shared_down,
           shared_expert_gate, linear_layer_idx):
    f32 = jnp.float32
    x2d = x[0]
    anw = attention_norm_weight[None, :]
    dtb = dt_bias[None, :]
    alog = A_log[None, :]

    nrow = S // RB
    qkv, z, g, beta = pl.pallas_call(
        _prologue_kernel,
        grid=(nrow,),
        in_specs=[
            pl.BlockSpec((RB, DIM), lambda i: (i, 0)),
            pl.BlockSpec((DIM, CONV_DIM), lambda i: (0, 0)),
            pl.BlockSpec((DIM, VAL_DIM), lambda i: (0, 0)),
            pl.BlockSpec((DIM, NV), lambda i: (0, 0)),
            pl.BlockSpec((DIM, NV), lambda i: (0, 0)),
            pl.BlockSpec((1, DIM), lambda i: (0, 0)),
            pl.BlockSpec((1, NV), lambda i: (0, 0)),
            pl.BlockSpec((1, NV), lambda i: (0, 0)),
        ],
        out_specs=[
            pl.BlockSpec((RB, CONV_DIM), lambda i: (i, 0)),
            pl.BlockSpec((RB, VAL_DIM), lambda i: (i, 0)),
            pl.BlockSpec((RB, NV), lambda i: (i, 0)),
            pl.BlockSpec((RB, NV), lambda i: (i, 0)),
        ],
        out_shape=[
            jax.ShapeDtypeStruct((S, CONV_DIM), f32),
            jax.ShapeDtypeStruct((S, VAL_DIM), f32),
            jax.ShapeDtypeStruct((S, NV), f32),
            jax.ShapeDtypeStruct((S, NV), f32),
        ],
    )(x2d, in_proj_qkv, in_proj_z, in_proj_a, in_proj_b, anw, dtb, alog)

    cs2 = conv_state[0].T
    wt = conv1d_weight.T

    qn, kn, v = pl.pallas_call(
        _conv_kernel,
        in_specs=[
            pl.BlockSpec((S, CONV_DIM), lambda: (0, 0)),
            pl.BlockSpec((KCONV - 1, CONV_DIM), lambda: (0, 0)),
            pl.BlockSpec((KCONV, CONV_DIM), lambda: (0, 0)),
        ],
        out_specs=[
            pl.BlockSpec((S, KEY_DIM), lambda: (0, 0)),
            pl.BlockSpec((S, KEY_DIM), lambda: (0, 0)),
            pl.BlockSpec((S, VAL_DIM), lambda: (0, 0)),
        ],
        out_shape=[
            jax.ShapeDtypeStruct((S, KEY_DIM), f32),
            jax.ShapeDtypeStruct((S, KEY_DIM), f32),
            jax.ShapeDtypeStruct((S, VAL_DIM), f32),
        ],
        scratch_shapes=[pltpu.VMEM((S + 8, CONV_DIM), f32)],
    )(qkv, cs2, wt)

    s0 = state[0].reshape(NV * DK, DV)
    o, snew = pl.pallas_call(
        _deltanet_kernel,
        grid=(NCHUNK,),
        in_specs=[
            pl.BlockSpec((CHUNK, KEY_DIM), lambda i: (i, 0)),
            pl.BlockSpec((CHUNK, KEY_DIM), lambda i: (i, 0)),
            pl.BlockSpec((CHUNK, VAL_DIM), lambda i: (i, 0)),
            pl.BlockSpec((CHUNK, NV), lambda i: (i, 0)),
            pl.BlockSpec((CHUNK, NV), lambda i: (i, 0)),
            pl.BlockSpec((NV * DK, DV), lambda i: (0, 0)),
        ],
        out_specs=[
            pl.BlockSpec((CHUNK, VAL_DIM), lambda i: (i, 0)),
            pl.BlockSpec((NV * DK, DV), lambda i: (0, 0)),
        ],
        out_shape=[
            jax.ShapeDtypeStruct((S, VAL_DIM), f32),
            jax.ShapeDtypeStruct((NV * DK, DV), f32),
        ],
        scratch_shapes=[pltpu.VMEM((NV * DK, DV), f32)],
        compiler_params=pltpu.CompilerParams(
            dimension_semantics=("arbitrary",)),
    )(qn, kn, v, g, beta, s0)

    x2b, h2, cw = pl.pallas_call(
        _epilogue_kernel,
        grid=(nrow,),
        in_specs=[
            pl.BlockSpec((RB, VAL_DIM), lambda i: (i, 0)),
            pl.BlockSpec((RB, VAL_DIM), lambda i: (i, 0)),
            pl.BlockSpec((RB, DIM), lambda i: (i, 0)),
            pl.BlockSpec((1, VAL_DIM), lambda i: (0, 0)),
            pl.BlockSpec((1, DIM), lambda i: (0, 0)),
            pl.BlockSpec((VAL_DIM, DIM), lambda i: (0, 0)),
            pl.BlockSpec((DIM, E), lambda i: (0, 0)),
            pl.BlockSpec((DIM, SHARED_I), lambda i: (0, 0)),
            pl.BlockSpec((DIM, SHARED_I), lambda i: (0, 0)),
            pl.BlockSpec((SHARED_I, DIM), lambda i: (0, 0)),
            pl.BlockSpec((DIM, 1), lambda i: (0, 0)),
        ],
        out_specs=[
            pl.BlockSpec((RB, DIM), lambda i: (i, 0)),
            pl.BlockSpec((RB, DIM), lambda i: (i, 0)),
            pl.BlockSpec((RB, E), lambda i: (i, 0)),
        ],
        out_shape=[
            jax.ShapeDtypeStruct((S, DIM), f32),
            jax.ShapeDtypeStruct((S, DIM), jnp.bfloat16),
            jax.ShapeDtypeStruct((S, E), f32),
        ],
    )(o, z, x2d, norm_weight[None, :], ffn_norm_weight[None, :],
      out_proj, router_weight.T, shared_gate, shared_up, shared_down,
      shared_expert_gate)

    x3 = pl.pallas_call(
        _expert_kernel,
        grid=(E,),
        in_specs=[
            pl.BlockSpec((S, DIM), lambda e: (0, 0)),
            pl.BlockSpec((1, 2 * MOE_I, DIM), lambda e: (e, 0, 0)),
            pl.BlockSpec((1, DIM, MOE_I), lambda e: (e, 0, 0)),
            pl.BlockSpec((S, E), lambda e: (0, 0)),
            pl.BlockSpec((S, DIM), lambda e: (0, 0)),
        ],
        out_specs=pl.BlockSpec((S, DIM), lambda e: (0, 0)),
        out_shape=jax.ShapeDtypeStruct((S, DIM), f32),
        scratch_shapes=[pltpu.VMEM((S, DIM), f32)],
        compiler_params=pltpu.CompilerParams(
            dimension_semantics=("arbitrary",)),
    )(h2, expert_gate_up, expert_down, cw, x2b)

    new_conv_state = qkv[S - (KCONV - 1):, :].T[None, :, :]
    return (x3[None, :, :], snew.reshape(1, NV, DK, DV), new_conv_state)
